# Initial kernel scaffold; baseline (speedup 1.0000x reference)
import jax
import jax.numpy as jnp
from jax.experimental import pallas as pl


def kernel(s1, s2, p1, p2, x, w, edge_index, W1, b1, WL, bL, Wh1, bh1, Wh2, bh2):
    raise NotImplementedError("write your pallas kernel here")


# speedup vs baseline: 4.0069x; 4.0069x over previous
import functools

import jax
import jax.numpy as jnp
from jax import lax
from jax.experimental import pallas as pl
from jax.experimental.pallas import tpu as pltpu
from jax.experimental.pallas import tpu_sc as plsc

D = 128
NC = 2
NS = 16
NW = NC * NS


def _prep_body(s1, s2, p1, p2, x, w, tsrc, tdst):
    tsrc[:, 0:D] = s1[...] + s2[...]
    tsrc[:, D:2 * D] = x[...]
    tsrc[:, 2 * D:3 * D] = w[...]
    tdst[:, 0:D] = p1[...] + p2[...]
    tdst[:, D:2 * D] = x[...]
    tdst[:, 2 * D:3 * D] = w[...]


def _build_tables(s1, s2, p1, p2, x, w):
    n = s1.shape[0]
    blk = 2000
    grid = (n // blk,)
    in_spec = pl.BlockSpec((blk, D), lambda i: (i, 0))
    out_spec = pl.BlockSpec((blk, 3 * D), lambda i: (i, 0))
    return pl.pallas_call(
        _prep_body,
        grid=grid,
        in_specs=[in_spec] * 6,
        out_specs=[out_spec, out_spec],
        out_shape=[jax.ShapeDtypeStruct((n, 3 * D), jnp.float32)] * 2,
    )(s1, s2, p1, p2, x, w)


def _sc_gather_combine(tsrc, tdst, src_idx, dst_idx):
    e = src_idx.shape[0]
    dt = tsrc.shape[1]
    ew = e // NW
    wnd = 80
    nch = ew // wnd
    mesh = plsc.VectorSubcoreMesh(core_axis_name="c", subcore_axis_name="s")

    @functools.partial(
        pl.kernel,
        out_type=jax.ShapeDtypeStruct((e, dt), jnp.float32),
        mesh=mesh,
        scratch_types=[
            pltpu.VMEM((ew,), jnp.int32),
            pltpu.VMEM((ew,), jnp.int32),
            pltpu.VMEM((wnd, dt), jnp.float32),
            pltpu.VMEM((wnd, dt), jnp.float32),
            pltpu.VMEM((wnd, dt), jnp.float32),
            pltpu.SemaphoreType.DMA,
            pltpu.SemaphoreType.DMA,
        ],
    )
    def k(tsrc_hbm, tdst_hbm, si_hbm, di_hbm, u_hbm,
          si_v, di_v, a_v, b_v, u_v, sem_a, sem_b):
        wid = lax.axis_index("s") * NC + lax.axis_index("c")
        base = wid * ew
        pltpu.sync_copy(si_hbm.at[pl.ds(base, ew)], si_v)
        pltpu.sync_copy(di_hbm.at[pl.ds(base, ew)], di_v)

        @pl.loop(0, nch)
        def _chunk(i):
            off = i * wnd
            ca = pltpu.async_copy(tsrc_hbm.at[si_v.at[pl.ds(off, wnd)]],
                                  a_v, sem_a)
            cb = pltpu.async_copy(tdst_hbm.at[di_v.at[pl.ds(off, wnd)]],
                                  b_v, sem_b)
            ca.wait()
            cb.wait()

            @pl.loop(0, wnd)
            def _row(r):
                for j in range(dt // 16):
                    slc = (pl.ds(r, 1), pl.ds(j * 16, 16))
                    av = a_v.at[*slc][...]
                    bv = b_v.at[*slc][...]
                    u_v.at[*slc][...] = av + bv if j < 8 else av * bv

            pltpu.sync_copy(u_v, u_hbm.at[pl.ds(base + off, wnd)])

    return k(tsrc, tdst, src_idx, dst_idx)


def _leaky(v):
    return jnp.maximum(v, 0.01 * v)


def _mlp_body(u, w1, b1, wl, wh1, wh2, cst, o):
    ub = u[...]
    z = jnp.dot(_leaky(ub[:, 0:D]), w1[...],
                preferred_element_type=jnp.float32) + b1[...]
    o[...] = (jnp.dot(_leaky(z), wl[...], preferred_element_type=jnp.float32)
              + jnp.dot(ub[:, D:2 * D], wh1[...],
                        preferred_element_type=jnp.float32)
              + jnp.dot(ub[:, 2 * D:3 * D], wh2[...],
                        preferred_element_type=jnp.float32)
              + cst[...])


def _tc_mlp(u, w1, b1, wl, wh1, wh2, cst):
    e = u.shape[0]
    blk = 2000
    grid = (e // blk,)
    full = lambda i: (0, 0)
    return pl.pallas_call(
        _mlp_body,
        grid=grid,
        in_specs=[
            pl.BlockSpec((blk, 3 * D), lambda i: (i, 0)),
            pl.BlockSpec((D, D), full),
            pl.BlockSpec((1, D), full),
            pl.BlockSpec((D, 1), full),
            pl.BlockSpec((D, 1), full),
            pl.BlockSpec((D, 1), full),
            pl.BlockSpec((1, 1), full),
        ],
        out_specs=pl.BlockSpec((blk, 1), lambda i: (i, 0)),
        out_shape=jax.ShapeDtypeStruct((e, 1), jnp.float32),
    )(u, w1, b1, wl, wh1, wh2, cst)


def kernel(s1, s2, p1, p2, x, w, edge_index, W1, b1, WL, bL, Wh1, bh1, Wh2, bh2):
    tsrc, tdst = _build_tables(s1, s2, p1, p2, x, w)
    src = edge_index[0].astype(jnp.int32)
    dst = edge_index[1].astype(jnp.int32)
    u = _sc_gather_combine(tsrc, tdst, src, dst)
    cst = (bL + bh1 + bh2).reshape(1, 1).astype(jnp.float32)
    return _tc_mlp(u, W1, b1.reshape(1, D), WL, Wh1, Wh2, cst)
